# baseline (speedup 1.0000x reference)
import functools

import jax
import jax.numpy as jnp
from jax import lax
from jax.experimental import pallas as pl
from jax.experimental.pallas import tpu as pltpu
from jax.experimental.pallas import tpu_sc as plsc

B, S, D = 4, 2048, 1024
NC, NS, L = 2, 16, 16
NW = NC * NS
S_PER_W = S // NW


def _tc_add_kernel(x_ref, t_ref, o_ref):
    o_ref[...] = x_ref[...] + t_ref[...][None, :, :]


def _kernel_tc(x, table):
    TS = 512
    return pl.pallas_call(
        _tc_add_kernel,
        grid=(S // TS,),
        in_specs=[
            pl.BlockSpec((B, TS, D), lambda s: (0, s, 0)),
            pl.BlockSpec((TS, D), lambda s: (s, 0)),
        ],
        out_specs=pl.BlockSpec((B, TS, D), lambda s: (0, s, 0)),
        out_shape=jax.ShapeDtypeStruct((B, S, D), x.dtype),
    )(x, table[:S])


_sc_mesh = plsc.VectorSubcoreMesh(core_axis_name="c", subcore_axis_name="s")

CHS = 8
NCH = S_PER_W // CHS
NSET = 3

NP = 2
NPC = B // NP

_scratch = (
    [pltpu.VMEM((B, CHS, D), jnp.float32) for _ in range(NSET)]
    + [pltpu.VMEM((CHS, D), jnp.float32) for _ in range(NSET)]
    + [pltpu.SemaphoreType.DMA for _ in range(NSET)]
    + [pltpu.SemaphoreType.DMA for _ in range(NSET * NPC)]
    + [pltpu.SemaphoreType.DMA for _ in range(NSET * NPC)]
)


@functools.partial(
    pl.kernel,
    mesh=_sc_mesh,
    out_type=jax.ShapeDtypeStruct((B, S, D), jnp.float32),
    scratch_types=_scratch,
)
def _sc_body(x_hbm, t_hbm, out_hbm, *scratch):
    xbufs = scratch[0:NSET]
    tbufs = scratch[NSET:2 * NSET]
    t_sems = scratch[2 * NSET:3 * NSET]
    in_sems = [scratch[3 * NSET + p * NPC:3 * NSET + (p + 1) * NPC]
               for p in range(NSET)]
    out_sems = [scratch[3 * NSET + NSET * NPC + p * NPC:
                        3 * NSET + NSET * NPC + (p + 1) * NPC]
                for p in range(NSET)]

    wid = lax.axis_index("s") * NC + lax.axis_index("c")
    base = wid * S_PER_W

    def fire_in(p, c):
        s0 = base + c * CHS
        pltpu.async_copy(t_hbm.at[pl.ds(s0, CHS)], tbufs[p], t_sems[p])
        for q in range(NPC):
            pltpu.async_copy(
                x_hbm.at[pl.ds(q * NP, NP), pl.ds(s0, CHS), :],
                xbufs[p].at[pl.ds(q * NP, NP)], in_sems[p][q])

    def wait_tbl(p):
        pltpu.make_async_copy(t_hbm.at[pl.ds(0, CHS)], tbufs[p],
                              t_sems[p]).wait()

    def wait_in(p, q):
        pltpu.make_async_copy(
            x_hbm.at[pl.ds(0, NP), pl.ds(0, CHS), :],
            xbufs[p].at[pl.ds(0, NP)], in_sems[p][q]).wait()

    def fire_out(p, q, c):
        s0 = base + c * CHS
        pltpu.async_copy(
            xbufs[p].at[pl.ds(q * NP, NP)],
            out_hbm.at[pl.ds(q * NP, NP), pl.ds(s0, CHS), :],
            out_sems[p][q])

    def wait_out(p, q):
        pltpu.make_async_copy(
            x_hbm.at[pl.ds(0, NP), pl.ds(0, CHS), :],
            xbufs[p].at[pl.ds(0, NP)], out_sems[p][q]).wait()

    def compute_piece(p, q):
        tb = tbufs[p]
        xb = xbufs[p]

        @plsc.parallel_loop(0, CHS, unroll=1)
        def row_add(i):
            def jhalf(jj, _):
                for j3 in range(32):
                    sl = pl.ds(jj * (32 * L) + j3 * L, L)
                    tv = tb[i, sl]
                    for b in range(q * NP, (q + 1) * NP):
                        plsc.addupdate(xb.at[b, i, sl], tv)
                return 0
            lax.fori_loop(0, D // (32 * L), jhalf, 0)

    for c in range(NSET):
        fire_in(c % NSET, c)
    pending_out = [False] * NSET
    for c in range(NCH):
        s = c % NSET
        wait_tbl(s)
        for q in range(NPC):
            wait_in(s, q)
            compute_piece(s, q)
            fire_out(s, q, c)
        pending_out[s] = True
        t = c + 2
        if c >= 1 and t < NCH:
            s2 = t % NSET
            if pending_out[s2]:
                for q in range(NPC):
                    wait_out(s2, q)
                pending_out[s2] = False
            fire_in(s2, t)
    for s in range(NSET):
        if pending_out[s]:
            for q in range(NPC):
                wait_out(s, q)


def _kernel_sc(x, table):
    return _sc_body(x, table[:S])


kernel = _kernel_sc


# speedup vs baseline: 1.0679x; 1.0679x over previous
import functools

import jax
import jax.numpy as jnp
from jax import lax
from jax.experimental import pallas as pl
from jax.experimental.pallas import tpu as pltpu
from jax.experimental.pallas import tpu_sc as plsc

B, S, D = 4, 2048, 1024
NC, NS, L = 2, 16, 16
NW = NC * NS
S_PER_W = S // NW


def _tc_add_kernel(x_ref, t_ref, o_ref):
    o_ref[...] = x_ref[...] + t_ref[...][None, :, :]


def _kernel_tc(x, table):
    TS = 512
    return pl.pallas_call(
        _tc_add_kernel,
        grid=(S // TS,),
        in_specs=[
            pl.BlockSpec((B, TS, D), lambda s: (0, s, 0)),
            pl.BlockSpec((TS, D), lambda s: (s, 0)),
        ],
        out_specs=pl.BlockSpec((B, TS, D), lambda s: (0, s, 0)),
        out_shape=jax.ShapeDtypeStruct((B, S, D), x.dtype),
    )(x, table[:S])


_sc_mesh = plsc.VectorSubcoreMesh(core_axis_name="c", subcore_axis_name="s")

CHS = 8
NCH = S_PER_W // CHS
NSET = 3

_scratch = (
    [pltpu.VMEM((B, CHS, D), jnp.float32) for _ in range(NSET)]
    + [pltpu.VMEM((CHS, D), jnp.float32) for _ in range(NSET)]
    + [pltpu.SemaphoreType.DMA for _ in range(2 * NSET)]
)


@functools.partial(
    pl.kernel,
    mesh=_sc_mesh,
    out_type=jax.ShapeDtypeStruct((B, S, D), jnp.float32),
    scratch_types=_scratch,
)
def _sc_body(x_hbm, t_hbm, out_hbm, *scratch):
    xbufs = scratch[0:NSET]
    tbufs = scratch[NSET:2 * NSET]
    in_sems = scratch[2 * NSET:3 * NSET]
    out_sems = scratch[3 * NSET:4 * NSET]

    wid = lax.axis_index("s") * NC + lax.axis_index("c")
    base = wid * S_PER_W

    def fire_in(p, c):
        s0 = base + c * CHS
        pltpu.async_copy(t_hbm.at[pl.ds(s0, CHS)], tbufs[p], in_sems[p])
        pltpu.async_copy(x_hbm.at[:, pl.ds(s0, CHS), :], xbufs[p], in_sems[p])

    def wait_in(p):
        pltpu.make_async_copy(t_hbm.at[pl.ds(0, CHS)], tbufs[p],
                              in_sems[p]).wait()
        pltpu.make_async_copy(x_hbm.at[:, pl.ds(0, CHS), :], xbufs[p],
                              in_sems[p]).wait()

    def fire_out(p, c):
        s0 = base + c * CHS
        pltpu.async_copy(xbufs[p], out_hbm.at[:, pl.ds(s0, CHS), :],
                         out_sems[p])

    def wait_out(p):
        pltpu.make_async_copy(x_hbm.at[:, pl.ds(0, CHS), :], xbufs[p],
                              out_sems[p]).wait()

    def compute(p):
        tb = tbufs[p]
        xb = xbufs[p]

        @plsc.parallel_loop(0, CHS, unroll=2)
        def row_add(i):
            for j in range(D // L):
                sl = pl.ds(j * L, L)
                tv = tb[i, sl]
                for b in range(B):
                    plsc.addupdate(xb.at[b, i, sl], tv)

    for c in range(NSET):
        fire_in(c % NSET, c)
    pending_out = [False] * NSET
    for c in range(NCH):
        s = c % NSET
        wait_in(s)
        compute(s)
        fire_out(s, c)
        pending_out[s] = True
        t = c + 2
        if c >= 1 and t < NCH:
            s2 = t % NSET
            if pending_out[s2]:
                wait_out(s2)
                pending_out[s2] = False
            fire_in(s2, t)
    for s in range(NSET):
        if pending_out[s]:
            wait_out(s)


def _kernel_sc(x, table):
    return _sc_body(x, table[:S])


kernel = _kernel_sc


# speedup vs baseline: 1.0685x; 1.0006x over previous
import functools

import jax
import jax.numpy as jnp
from jax import lax
from jax.experimental import pallas as pl
from jax.experimental.pallas import tpu as pltpu
from jax.experimental.pallas import tpu_sc as plsc

B, S, D = 4, 2048, 1024
NC, NS, L = 2, 16, 16
NW = NC * NS
S_PER_W = S // NW


def _tc_add_kernel(x_ref, t_ref, o_ref):
    o_ref[...] = x_ref[...] + t_ref[...][None, :, :]


def _kernel_tc(x, table):
    TS = 512
    return pl.pallas_call(
        _tc_add_kernel,
        grid=(S // TS,),
        in_specs=[
            pl.BlockSpec((B, TS, D), lambda s: (0, s, 0)),
            pl.BlockSpec((TS, D), lambda s: (s, 0)),
        ],
        out_specs=pl.BlockSpec((B, TS, D), lambda s: (0, s, 0)),
        out_shape=jax.ShapeDtypeStruct((B, S, D), x.dtype),
    )(x, table[:S])


_sc_mesh = plsc.VectorSubcoreMesh(core_axis_name="c", subcore_axis_name="s")

CHS = 8
NCH = S_PER_W // CHS
NSET = 3

_scratch = (
    [pltpu.VMEM((B, CHS, D), jnp.float32) for _ in range(NSET)]
    + [pltpu.VMEM((CHS, D), jnp.float32) for _ in range(NSET)]
    + [pltpu.SemaphoreType.DMA for _ in range(2 * NSET)]
)


@functools.partial(
    pl.kernel,
    mesh=_sc_mesh,
    out_type=jax.ShapeDtypeStruct((B, S, D), jnp.float32),
    scratch_types=_scratch,
)
def _sc_body(x_hbm, t_hbm, out_hbm, *scratch):
    xbufs = scratch[0:NSET]
    tbufs = scratch[NSET:2 * NSET]
    in_sems = scratch[2 * NSET:3 * NSET]
    out_sems = scratch[3 * NSET:4 * NSET]

    wid = lax.axis_index("s") * NC + lax.axis_index("c")
    base = wid * S_PER_W

    def fire_in(p, c):
        s0 = base + c * CHS
        pltpu.async_copy(t_hbm.at[pl.ds(s0, CHS)], tbufs[p], in_sems[p])
        pltpu.async_copy(x_hbm.at[:, pl.ds(s0, CHS), :], xbufs[p], in_sems[p])

    def wait_in(p):
        pltpu.make_async_copy(t_hbm.at[pl.ds(0, CHS)], tbufs[p],
                              in_sems[p]).wait()
        pltpu.make_async_copy(x_hbm.at[:, pl.ds(0, CHS), :], xbufs[p],
                              in_sems[p]).wait()

    def fire_out(p, c):
        s0 = base + c * CHS
        pltpu.async_copy(xbufs[p], out_hbm.at[:, pl.ds(s0, CHS), :],
                         out_sems[p])

    def wait_out(p):
        pltpu.make_async_copy(x_hbm.at[:, pl.ds(0, CHS), :], xbufs[p],
                              out_sems[p]).wait()

    def compute(p):
        tb = tbufs[p]
        xb = xbufs[p]

        @plsc.parallel_loop(0, CHS, unroll=2)
        def row_add(i):
            for j in range(D // L):
                sl = pl.ds(j * L, L)
                tv = tb[i, sl]
                for b in range(B):
                    plsc.addupdate(xb.at[b, i, sl], tv)

    for c in range(NSET):
        fire_in(c % NSET, c)
    pending_out = [False] * NSET
    for c in range(NCH):
        s = c % NSET
        wait_in(s)
        compute(s)
        fire_out(s, c)
        pending_out[s] = True
        t = c + 2
        if c >= 1 and t >= NSET and t < NCH:
            s2 = t % NSET
            if pending_out[s2]:
                wait_out(s2)
                pending_out[s2] = False
            fire_in(s2, t)
    for s in range(NSET):
        if pending_out[s]:
            wait_out(s)


def _kernel_sc(x, table):
    return _sc_body(x, table[:S])


kernel = _kernel_sc
